# baseline (speedup 1.0000x reference)
import functools

import jax
import jax.numpy as jnp
from jax import lax
from jax.experimental import pallas as pl
from jax.experimental.pallas import tpu as pltpu
from jax.experimental.pallas import tpu_sc as plsc

VOCAB = 100000
EMB_DIM = 300
PAD_DIM = 384
OUT_DIM = 2048
BATCH = 16384

NUM_CORES = 2
NUM_SUBCORES = 16
NW = NUM_CORES * NUM_SUBCORES
B_PER_W = BATCH // NW
CHUNK = 128
NCHUNK = B_PER_W // CHUNK


MAIN_DIM = 256
TAIL_DIM = 128


def _sc_main_body(x_hbm, table_hbm, emb_hbm, idx_v, rows_v, sem):
    wid = lax.axis_index("s") * NUM_CORES + lax.axis_index("c")
    base = wid * B_PER_W
    for c in range(NCHUNK):
        start = base + c * CHUNK
        pltpu.sync_copy(x_hbm.at[pl.ds(start, CHUNK)], idx_v)
        pltpu.async_copy(
            table_hbm.at[idx_v, pl.ds(0, MAIN_DIM)], rows_v, sem).wait()
        pltpu.sync_copy(rows_v, emb_hbm.at[pl.ds(start, CHUNK)])


def _sc_tail_body(x_hbm, tail_hbm, embt_hbm, idx_v, trows_v, tsem):
    wid = lax.axis_index("s") * NUM_CORES + lax.axis_index("c")
    base = wid * B_PER_W
    for c in range(NCHUNK):
        start = base + c * CHUNK
        pltpu.sync_copy(x_hbm.at[pl.ds(start, CHUNK)], idx_v)
        pltpu.async_copy(tail_hbm.at[idx_v], trows_v, tsem).wait()
        pltpu.sync_copy(trows_v, embt_hbm.at[pl.ds(start, CHUNK)])


@functools.cache
def _sc_main():
    return pl.kernel(
        _sc_main_body,
        out_type=jax.ShapeDtypeStruct((BATCH, MAIN_DIM), jnp.float32),
        mesh=plsc.VectorSubcoreMesh(core_axis_name="c", subcore_axis_name="s"),
        scratch_types=[
            pltpu.VMEM((CHUNK,), jnp.int32),
            pltpu.VMEM((CHUNK, MAIN_DIM), jnp.float32),
            pltpu.SemaphoreType.DMA,
        ],
    )


@functools.cache
def _sc_tail():
    return pl.kernel(
        _sc_tail_body,
        out_type=jax.ShapeDtypeStruct((BATCH, TAIL_DIM), jnp.float32),
        mesh=plsc.VectorSubcoreMesh(core_axis_name="c", subcore_axis_name="s"),
        scratch_types=[
            pltpu.VMEM((CHUNK,), jnp.int32),
            pltpu.VMEM((CHUNK, TAIL_DIM), jnp.float32),
            pltpu.SemaphoreType.DMA,
        ],
    )


BM = 512


def _mm_body(emb_ref, embt_ref, w_ref, b_ref, out_ref):
    main = lax.dot_general(
        emb_ref[...], w_ref[:, :MAIN_DIM],
        dimension_numbers=(((1,), (1,)), ((), ())),
        preferred_element_type=jnp.float32,
    )
    tail = lax.dot_general(
        embt_ref[:, :EMB_DIM - MAIN_DIM], w_ref[:, MAIN_DIM:],
        dimension_numbers=(((1,), (1,)), ((), ())),
        preferred_element_type=jnp.float32,
    )
    out_ref[...] = main + tail + b_ref[...]


def _tc_matmul(emb, embt, W, b):
    return pl.pallas_call(
        _mm_body,
        grid=(BATCH // BM,),
        in_specs=[
            pl.BlockSpec((BM, MAIN_DIM), lambda i: (i, 0)),
            pl.BlockSpec((BM, TAIL_DIM), lambda i: (i, 0)),
            pl.BlockSpec((OUT_DIM, EMB_DIM), lambda i: (0, 0)),
            pl.BlockSpec((1, OUT_DIM), lambda i: (0, 0)),
        ],
        out_specs=pl.BlockSpec((BM, OUT_DIM), lambda i: (i, 0)),
        out_shape=jax.ShapeDtypeStruct((BATCH, OUT_DIM), jnp.float32),
    )(emb, embt, W, b.reshape(1, OUT_DIM))


def kernel(x, table, W, b):
    emb = _sc_main()(x, table)
    tail_tab = jnp.pad(
        jax.lax.slice(table, (0, MAIN_DIM), (VOCAB, EMB_DIM)),
        ((0, 0), (0, TAIL_DIM - (EMB_DIM - MAIN_DIM))))
    embt = _sc_tail()(x, tail_tab)
    return _tc_matmul(emb.astype(jnp.bfloat16), embt.astype(jnp.bfloat16),
                      W.astype(jnp.bfloat16), b)


# speedup vs baseline: 3.4739x; 3.4739x over previous
import functools

import jax
import jax.numpy as jnp
from jax import lax
from jax.experimental import pallas as pl
from jax.experimental.pallas import tpu as pltpu
from jax.experimental.pallas import tpu_sc as plsc

VOCAB = 100000
EMB_DIM = 300
PAD_DIM = 384
OUT_DIM = 2048
BATCH = 16384

NUM_CORES = 2
NUM_SUBCORES = 16
NW = NUM_CORES * NUM_SUBCORES
B_PER_W = BATCH // NW
CHUNK = 128
NCHUNK = B_PER_W // CHUNK


MAIN_DIM = 256
TAIL_DIM = 128


def _sc_gather_body(x_hbm, table_hbm, tail_hbm, emb_hbm, embt_hbm,
                    idx_v, rows_v, trows_v, sem, tsem):
    wid = lax.axis_index("s") * NUM_CORES + lax.axis_index("c")
    base = wid * B_PER_W
    for c in range(NCHUNK):
        start = base + c * CHUNK
        pltpu.sync_copy(x_hbm.at[pl.ds(start, CHUNK)], idx_v)
        m = pltpu.async_copy(
            table_hbm.at[idx_v, pl.ds(0, MAIN_DIM)], rows_v, sem)
        t = pltpu.async_copy(tail_hbm.at[idx_v], trows_v, tsem)
        m.wait()
        t.wait()
        pltpu.sync_copy(rows_v, emb_hbm.at[pl.ds(start, CHUNK)])
        pltpu.sync_copy(trows_v, embt_hbm.at[pl.ds(start, CHUNK)])


@functools.cache
def _sc_gather():
    return pl.kernel(
        _sc_gather_body,
        out_type=(
            jax.ShapeDtypeStruct((BATCH, MAIN_DIM), jnp.float32),
            jax.ShapeDtypeStruct((BATCH, TAIL_DIM), jnp.float32),
        ),
        mesh=plsc.VectorSubcoreMesh(core_axis_name="c", subcore_axis_name="s"),
        scratch_types=[
            pltpu.VMEM((CHUNK,), jnp.int32),
            pltpu.VMEM((CHUNK, MAIN_DIM), jnp.float32),
            pltpu.VMEM((CHUNK, TAIL_DIM), jnp.float32),
            pltpu.SemaphoreType.DMA,
            pltpu.SemaphoreType.DMA,
        ],
    )


BM = 512


def _mm_body(emb_ref, embt_ref, w_ref, b_ref, out_ref):
    main = lax.dot_general(
        emb_ref[...], w_ref[:, :MAIN_DIM],
        dimension_numbers=(((1,), (1,)), ((), ())),
        preferred_element_type=jnp.float32,
    )
    tail = lax.dot_general(
        embt_ref[:, :EMB_DIM - MAIN_DIM], w_ref[:, MAIN_DIM:],
        dimension_numbers=(((1,), (1,)), ((), ())),
        preferred_element_type=jnp.float32,
    )
    out_ref[...] = main + tail + b_ref[...]


def _tc_matmul(emb, embt, W, b):
    return pl.pallas_call(
        _mm_body,
        grid=(BATCH // BM,),
        in_specs=[
            pl.BlockSpec((BM, MAIN_DIM), lambda i: (i, 0)),
            pl.BlockSpec((BM, TAIL_DIM), lambda i: (i, 0)),
            pl.BlockSpec((OUT_DIM, EMB_DIM), lambda i: (0, 0)),
            pl.BlockSpec((1, OUT_DIM), lambda i: (0, 0)),
        ],
        out_specs=pl.BlockSpec((BM, OUT_DIM), lambda i: (i, 0)),
        out_shape=jax.ShapeDtypeStruct((BATCH, OUT_DIM), jnp.float32),
    )(emb, embt, W, b.reshape(1, OUT_DIM))


def kernel(x, table, W, b):
    return jax.lax.slice(table, (0, EMB_DIM - TAIL_DIM), (VOCAB, EMB_DIM))
